# baseline (speedup 1.0000x reference)
import functools

import jax
import jax.numpy as jnp
from jax import lax
from jax.experimental import pallas as pl
from jax.experimental.pallas import tpu as pltpu
from jax.experimental.pallas import tpu_sc as plsc

_LANES = 16


def _g_body(h_ref, w_ref, b_ref, g_ref):
    g = jnp.mean(h_ref[...], axis=1)
    g_ref[...] = (
        lax.dot_general(g, w_ref[...], (((1,), (1,)), ((), ())),
                        precision=lax.Precision.HIGHEST)
        + b_ref[...]
    )


def _compute_g(H_lr, W, b):
    Bsz, _, D = H_lr.shape
    return pl.pallas_call(
        _g_body,
        out_shape=jax.ShapeDtypeStruct((Bsz, D), jnp.float32),
    )(H_lr, W, b.reshape(1, D))


@functools.lru_cache(maxsize=None)
def _make_sc_add(hr_n, Bsz, D):
    info = plsc.get_sparse_core_info()
    nc, ns = info.num_cores, info.num_subcores
    nw = nc * ns
    max_r = max(8, (42 * 1024) // (D * 4))
    R = 8
    for cand in range(8, max_r + 1, 8):
        if hr_n % cand == 0:
            R = cand
    bpw = hr_n // (R * nw)
    rows_w = bpw * R
    main_rows = rows_w * nw
    rem8 = (hr_n - main_rows) // 8
    nsb = bpw // 2
    tail_blk = bpw % 2
    nchunk = D // _LANES

    mesh = plsc.VectorSubcoreMesh(core_axis_name="c", subcore_axis_name="s")

    @functools.partial(
        pl.kernel,
        out_type=jax.ShapeDtypeStruct((Bsz, hr_n, D), jnp.float32),
        mesh=mesh,
        scratch_types=[
            pltpu.VMEM((Bsz, D), jnp.float32),
            pltpu.VMEM((2, 2 * R, D), jnp.float32),
            pltpu.VMEM((2, Bsz, R, D), jnp.float32),
            pltpu.SemaphoreType.DMA,
            pltpu.SemaphoreType.DMA,
            pltpu.SemaphoreType.DMA,
            pltpu.SemaphoreType.DMA,
        ],
    )
    def sc_add(table_hbm, g_hbm, out_hbm, g_v, in_v, out_v, si0, si1, so0, so1):
        wid = lax.axis_index("s") * nc + lax.axis_index("c")
        base = wid * rows_w
        pltpu.sync_copy(g_hbm, g_v)
        sin = (si0, si1)
        sout = (so0, so1)

        def in_cp(sb, slot):
            row0 = base + sb * (2 * R)
            return pltpu.make_async_copy(
                table_hbm.at[pl.ds(row0, 2 * R)], in_v.at[slot], sin[slot])

        def out_cp(tb, oslot):
            row0 = base + tb * R
            return pltpu.make_async_copy(
                out_v.at[oslot], out_hbm.at[:, pl.ds(row0, R)], sout[oslot])

        gs = [[g_v[bb, pl.ds(_LANES * j, _LANES)] for j in range(nchunk)]
              for bb in range(Bsz)]

        def compute(islot, ibase, oslot, rows):
            def row_body(r, c):
                vin = [in_v[islot, ibase + r, pl.ds(_LANES * j, _LANES)]
                       for j in range(nchunk)]
                for bb in range(Bsz):
                    for j in range(nchunk):
                        sl = pl.ds(_LANES * j, _LANES)
                        out_v[oslot, bb, r, sl] = vin[j] + gs[bb][j]
                return c

            lax.fori_loop(0, rows, row_body, 0)

        def sb_step(sb, slot, first):
            in_cp(sb, slot).wait()
            for h in range(2):
                if not first:
                    out_cp(0, h).wait()
                compute(slot, h * R, h, R)
                out_cp(2 * sb + h, h).start()

            @pl.when(sb + 2 < nsb)
            def _():
                in_cp(sb + 2, slot).start()

        in_cp(0, 0).start()
        in_cp(1, 1).start()
        sb_step(0, 0, True)
        sb_step(1, 1, False)

        def sb_body(k, c):
            sb_step(2 * k, 0, False)
            sb_step(2 * k + 1, 1, False)
            return c

        lax.fori_loop(1, nsb // 2, sb_body, 0)
        if nsb % 2:
            sb_step(nsb - 1, (nsb - 1) % 2, False)

        if tail_blk:
            tb = bpw - 1
            pltpu.sync_copy(table_hbm.at[pl.ds(base + tb * R, R)],
                            in_v.at[0, pl.ds(0, R)])
            out_cp(0, 0).wait()
            compute(0, 0, 0, R)
            out_cp(tb, 0).start()

        out_cp(0, 0).wait()
        out_cp(0, 1).wait()

        if rem8:
            @pl.when(wid < rem8)
            def _():
                row0 = main_rows + wid * 8
                pltpu.sync_copy(table_hbm.at[pl.ds(row0, 8)],
                                in_v.at[0, pl.ds(0, 8)])
                compute(0, 0, 0, 8)
                for bb in range(Bsz):
                    pltpu.sync_copy(out_v.at[0, bb, pl.ds(0, 8)],
                                    out_hbm.at[bb, pl.ds(row0, 8)])

    return sc_add


def kernel(H_lr, emb_table, W, b):
    hr_n, D = emb_table.shape
    Bsz = H_lr.shape[0]
    g = _compute_g(H_lr, W, b)
    return _make_sc_add(hr_n, Bsz, D)(emb_table, g)


# speedup vs baseline: 1.0247x; 1.0247x over previous
import functools

import jax
import jax.numpy as jnp
from jax import lax
from jax.experimental import pallas as pl
from jax.experimental.pallas import tpu as pltpu
from jax.experimental.pallas import tpu_sc as plsc

_LANES = 16


def _g_body(h_ref, w_ref, b_ref, g_ref):
    g = jnp.mean(h_ref[...], axis=1)
    g_ref[...] = (
        lax.dot_general(g, w_ref[...], (((1,), (1,)), ((), ())),
                        precision=lax.Precision.HIGHEST)
        + b_ref[...]
    )


def _compute_g(H_lr, W, b):
    Bsz, _, D = H_lr.shape
    return pl.pallas_call(
        _g_body,
        out_shape=jax.ShapeDtypeStruct((Bsz, D), jnp.float32),
    )(H_lr, W, b.reshape(1, D))


@functools.lru_cache(maxsize=None)
def _make_sc_add(hr_n, Bsz, D):
    info = plsc.get_sparse_core_info()
    nc, ns = info.num_cores, info.num_subcores
    nw = nc * ns
    max_r = max(8, (42 * 1024) // (D * 4))
    R = 8
    for cand in range(8, max_r + 1, 8):
        if hr_n % cand == 0:
            R = cand
    bpw = hr_n // (R * nw)
    main_rows = bpw * nw * R
    rem8 = (hr_n - main_rows) // 8
    npairs = bpw // 2
    nchunk = D // _LANES

    mesh = plsc.VectorSubcoreMesh(core_axis_name="c", subcore_axis_name="s")

    @functools.partial(
        pl.kernel,
        out_type=jax.ShapeDtypeStruct((Bsz, hr_n, D), jnp.float32),
        mesh=mesh,
        scratch_types=[
            pltpu.VMEM((Bsz, D), jnp.float32),
            pltpu.VMEM((2, R, D), jnp.float32),
            pltpu.VMEM((2, Bsz, R, D), jnp.float32),
            pltpu.SemaphoreType.DMA,
            pltpu.SemaphoreType.DMA,
            pltpu.SemaphoreType.DMA,
            pltpu.SemaphoreType.DMA,
        ],
    )
    def sc_add(table_hbm, g_hbm, out_hbm, g_v, in_v, out_v, si0, si1, so0, so1):
        wid = lax.axis_index("s") * nc + lax.axis_index("c")
        pltpu.sync_copy(g_hbm, g_v)
        sin = (si0, si1)
        sout = (so0, so1)

        def in_cp(t, slot):
            row0 = (wid + t * nw) * R
            return pltpu.make_async_copy(
                table_hbm.at[pl.ds(row0, R)], in_v.at[slot], sin[slot])

        def out_cp(t, slot):
            row0 = (wid + t * nw) * R
            return pltpu.make_async_copy(
                out_v.at[slot], out_hbm.at[:, pl.ds(row0, R)], sout[slot])

        gs = [[g_v[bb, pl.ds(_LANES * j, _LANES)] for j in range(nchunk)]
              for bb in range(Bsz)]

        def compute(slot, rows):
            def row_body(r, c, slot=slot):
                vin = [in_v[slot, r, pl.ds(_LANES * j, _LANES)]
                       for j in range(nchunk)]
                for bb in range(Bsz):
                    for j in range(nchunk):
                        sl = pl.ds(_LANES * j, _LANES)
                        out_v[slot, bb, r, sl] = vin[j] + gs[bb][j]
                return c

            lax.fori_loop(0, rows, row_body, 0)

        def step(t, slot, first):
            in_cp(t, slot).wait()
            if isinstance(t, int):
                if t + 2 < bpw:
                    in_cp(t + 2, slot).start()
            else:
                @pl.when(t + 2 < bpw)
                def _():
                    in_cp(t + 2, slot).start()
            if not first:
                out_cp(t, slot).wait()
            compute(slot, R)
            out_cp(t, slot).start()

        in_cp(0, 0).start()
        if bpw > 1:
            in_cp(1, 1).start()
        step(0, 0, True)
        if bpw > 1:
            step(1, 1, True)

        def pair_body(k, c):
            t = 2 * k
            step(t, 0, False)
            step(t + 1, 1, False)
            return c

        lax.fori_loop(1, npairs, pair_body, 0)

        if bpw % 2:
            step(bpw - 1, 0, bpw == 1)

        out_cp(0, 0).wait()
        if bpw > 1:
            out_cp(0, 1).wait()

        if rem8:
            @pl.when(wid < rem8)
            def _():
                row0 = main_rows + wid * 8
                pltpu.sync_copy(table_hbm.at[pl.ds(row0, 8)],
                                in_v.at[0, pl.ds(0, 8)])
                compute(0, 8)
                for bb in range(Bsz):
                    pltpu.sync_copy(out_v.at[0, bb, pl.ds(0, 8)],
                                    out_hbm.at[bb, pl.ds(row0, 8)])

    return sc_add


def kernel(H_lr, emb_table, W, b):
    hr_n, D = emb_table.shape
    Bsz = H_lr.shape[0]
    g = _compute_g(H_lr, W, b)
    return _make_sc_add(hr_n, Bsz, D)(emb_table, g)
